# Initial kernel scaffold; baseline (speedup 1.0000x reference)
import jax
import jax.numpy as jnp
from jax.experimental import pallas as pl


def kernel(x, edge_index, batch, W_gat, att_src, att_dst, b_gat, W2_rel, W2_root, b2, W3_l, b3, W3_r, W4_rel, W4_root, b4, W_fc1, b_fc1, W_fc2, b_fc2):
    raise NotImplementedError("write your pallas kernel here")


# speedup vs baseline: 1.0347x; 1.0347x over previous
import jax
import jax.numpy as jnp
from jax import lax
from jax.experimental import pallas as pl
from jax.experimental.pallas import tpu as pltpu

N = 10000
E = 320000
EG = 330000
HEADS = 8
HID = 64
F = HEADS * HID
G = 128
EB = 256
EBG = 80
_F32 = jnp.float32


def _d1_body(x_ref, wg_ref, asm_ref, adm_ref, h_ref, as_ref, ad_ref):
    h = jnp.dot(x_ref[...], wg_ref[...], preferred_element_type=_F32)
    h_ref[...] = h
    as_ref[...] = jnp.dot(h, asm_ref[...], preferred_element_type=_F32)
    ad_ref[...] = jnp.dot(h, adm_ref[...], preferred_element_type=_F32)


def _gat_body(src_ref, dstc_ref, dstr_ref, h_ref, as_ref, ad_ref,
              num, den):
    step = pl.program_id(0)

    @pl.when(step == 0)
    def _():
        num[...] = jnp.zeros_like(num)
        den[...] = jnp.zeros_like(den)

    src = src_ref[0]
    dstc = dstc_ref[0]
    dstr = dstr_ref[0]

    iota_en = lax.broadcasted_iota(jnp.int32, (EBG, N), 1)
    s_oh = jnp.where(src == iota_en, 1.0, 0.0)
    d_oh = jnp.where(dstc == iota_en, 1.0, 0.0)
    iota_ne = lax.broadcasted_iota(jnp.int32, (N, EBG), 0)
    d_ohT = jnp.where(iota_ne == dstr, 1.0, 0.0)

    hsrc = jnp.dot(s_oh, h_ref[...], preferred_element_type=_F32)
    ase = jnp.dot(s_oh, as_ref[...], preferred_element_type=_F32)
    ade = jnp.dot(d_oh, ad_ref[...], preferred_element_type=_F32)
    z = ase + ade
    e = jnp.where(z > 0, z, 0.2 * z)
    w = jnp.exp(e)

    den[...] += jnp.dot(d_ohT, w, preferred_element_type=_F32)
    parts = [hsrc[:, hh * HID:(hh + 1) * HID] * w[:, hh:hh + 1]
             for hh in range(HEADS)]
    hw = jnp.concatenate(parts, axis=1)
    num[...] += jnp.dot(d_ohT, hw, preferred_element_type=_F32)


def _seg_body(src_ref, dstr_ref, y_ref, out_ref, cnt_ref):
    step = pl.program_id(0)

    @pl.when(step == 0)
    def _():
        out_ref[...] = jnp.zeros_like(out_ref)
        cnt_ref[...] = jnp.zeros_like(cnt_ref)

    src = src_ref[0]
    dstr = dstr_ref[0]
    iota_en = lax.broadcasted_iota(jnp.int32, (EB, N), 1)
    s_oh = jnp.where(src == iota_en, 1.0, 0.0)
    iota_ne = lax.broadcasted_iota(jnp.int32, (N, EB), 0)
    d_ohT = jnp.where(iota_ne == dstr, 1.0, 0.0)
    hsrc = jnp.dot(s_oh, y_ref[...], preferred_element_type=_F32)
    out_ref[...] += jnp.dot(d_ohT, hsrc, preferred_element_type=_F32)
    cnt_ref[...] += jnp.sum(d_ohT, axis=1, keepdims=True)


def _d2_body(num_ref, den_ref, bg_ref, wrel_ref, wroot_ref, b_ref, y_ref, r_ref):
    dd = den_ref[...]
    outs = [num_ref[:, hh * HID:(hh + 1) * HID] / dd[:, hh:hh + 1]
            for hh in range(HEADS)]
    h1 = jnp.maximum(jnp.concatenate(outs, axis=1) + bg_ref[...], 0.0)
    y_ref[...] = jnp.dot(h1, wrel_ref[...], preferred_element_type=_F32)
    r_ref[...] = jnp.dot(h1, wroot_ref[...], preferred_element_type=_F32) + b_ref[...]


def _d3_body(agg_ref, r_ref, wl_ref, wr_ref, b3_ref, y_ref, rn_ref):
    h2 = jnp.maximum(agg_ref[...] + r_ref[...], 0.0)
    y_ref[...] = jnp.dot(h2, wl_ref[...], preferred_element_type=_F32)
    rn_ref[...] = jnp.dot(h2, wr_ref[...], preferred_element_type=_F32) + b3_ref[...]


def _d4_body(s3_ref, cnt_ref, r3_ref, wrel_ref, wroot_ref, b4_ref, y_ref, r_ref):
    mean = s3_ref[...] / jnp.maximum(cnt_ref[...], 1.0)
    h3 = jnp.maximum(mean + r3_ref[...], 0.0)
    y_ref[...] = jnp.dot(h3, wrel_ref[...], preferred_element_type=_F32)
    r_ref[...] = jnp.dot(h3, wroot_ref[...], preferred_element_type=_F32) + b4_ref[...]


def _pool_body(s4_ref, r4_ref, batch_ref, w1_ref, b1_ref, w2_ref, b2_ref, out_ref):
    h4 = jnp.maximum(s4_ref[...] + r4_ref[...], 0.0)
    iota_gn = lax.broadcasted_iota(jnp.int32, (G, N), 0)
    b_oh = jnp.where(iota_gn == batch_ref[...], 1.0, 0.0)
    sums = jnp.dot(b_oh, h4, preferred_element_type=_F32)
    cnts = jnp.sum(b_oh, axis=1, keepdims=True)
    g = sums / jnp.maximum(cnts, 1.0)
    g1 = jnp.maximum(jnp.dot(g, w1_ref[...], preferred_element_type=_F32)
                     + b1_ref[...], 0.0)
    out_ref[...] = jnp.dot(g1, w2_ref[...], preferred_element_type=_F32) + b2_ref[...]


def kernel(x, edge_index, batch, W_gat, att_src, att_dst, b_gat, W2_rel,
           W2_root, b2, W3_l, b3, W3_r, W4_rel, W4_root, b4, W_fc1, b_fc1,
           W_fc2, b_fc2):
    loop = jnp.arange(N, dtype=edge_index.dtype)
    srcg = jnp.concatenate([edge_index[0], loop])
    dstg = jnp.concatenate([edge_index[1], loop])
    nbg = EG // EBG
    srcg_c = srcg.reshape(nbg, EBG, 1)
    dstg_c = dstg.reshape(nbg, EBG, 1)
    dstg_r = dstg.reshape(nbg, 1, EBG)
    nb = E // EB
    src_c = edge_index[0].reshape(nb, EB, 1)
    dst_r = edge_index[1].reshape(nb, 1, EB)
    batch_r = batch.reshape(1, N)

    eye = jnp.eye(HEADS, dtype=_F32)
    asm = (att_src[0][:, :, None] * eye[:, None, :]).reshape(F, HEADS)
    adm = (att_dst[0][:, :, None] * eye[:, None, :]).reshape(F, HEADS)

    bgr = b_gat.reshape(1, F)
    b2r = b2.reshape(1, HID)
    b3r = b3.reshape(1, HID)
    b4r = b4.reshape(1, HID)
    bf1 = b_fc1.reshape(1, HID)
    bf2 = b_fc2.reshape(1, -1)

    full = lambda shape: pl.BlockSpec(shape, lambda *_: (0,) * len(shape))
    eb3 = lambda a, b: pl.BlockSpec((1, a, b), lambda i: (i, 0, 0))

    h, a_s, a_d = pl.pallas_call(
        _d1_body,
        out_shape=[jax.ShapeDtypeStruct((N, F), _F32),
                   jax.ShapeDtypeStruct((N, HEADS), _F32),
                   jax.ShapeDtypeStruct((N, HEADS), _F32)],
    )(x, W_gat, asm, adm)

    num, den = pl.pallas_call(
        _gat_body,
        grid=(nbg,),
        in_specs=[eb3(EBG, 1), eb3(EBG, 1), eb3(1, EBG),
                  full((N, F)), full((N, HEADS)), full((N, HEADS))],
        out_specs=[full((N, F)), full((N, HEADS))],
        out_shape=[jax.ShapeDtypeStruct((N, F), _F32),
                   jax.ShapeDtypeStruct((N, HEADS), _F32)],
    )(srcg_c, dstg_c, dstg_r, h, a_s, a_d)

    def seg(y):
        return pl.pallas_call(
            _seg_body,
            grid=(nb,),
            in_specs=[eb3(EB, 1), eb3(1, EB), full((N, HID))],
            out_specs=[full((N, HID)), full((N, 1))],
            out_shape=[jax.ShapeDtypeStruct((N, HID), _F32),
                       jax.ShapeDtypeStruct((N, 1), _F32)],
        )(src_c, dst_r, y)

    y2, r2 = pl.pallas_call(
        _d2_body,
        out_shape=[jax.ShapeDtypeStruct((N, HID), _F32),
                   jax.ShapeDtypeStruct((N, HID), _F32)],
    )(num, den, bgr, W2_rel, W2_root, b2r)
    agg2, _ = seg(y2)

    y3, r3 = pl.pallas_call(
        _d3_body,
        out_shape=[jax.ShapeDtypeStruct((N, HID), _F32),
                   jax.ShapeDtypeStruct((N, HID), _F32)],
    )(agg2, r2, W3_l, W3_r, b3r)
    s3, cnt = seg(y3)

    y4, r4 = pl.pallas_call(
        _d4_body,
        out_shape=[jax.ShapeDtypeStruct((N, HID), _F32),
                   jax.ShapeDtypeStruct((N, HID), _F32)],
    )(s3, cnt, r3, W4_rel, W4_root, b4r)
    s4, _ = seg(y4)

    out = pl.pallas_call(
        _pool_body,
        out_shape=jax.ShapeDtypeStruct((G, b_fc2.shape[0]), _F32),
    )(s4, r4, batch_r, W_fc1, bf1, W_fc2, bf2)
    return out
